# Initial kernel scaffold; baseline (speedup 1.0000x reference)
import jax
import jax.numpy as jnp
from jax import lax
from jax.experimental import pallas as pl
from jax.experimental.pallas import tpu as pltpu
from jax.experimental.pallas import tpu_sc as plsc

_N = 10000
_E = 320000
_H = 128

_NC = 2
_NS = 16
_NW = _NC * _NS
_EPT = _E // _NW
_EB = 80
_NB = _EPT // _EB
_RPT = _N // _NS

_RB = 2000
_EBT = 4000

_mesh = plsc.VectorSubcoreMesh(core_axis_name="c", subcore_axis_name="s")


def _wid():
    return lax.axis_index("s") * _NC + lax.axis_index("c")


def _deg_body(dst_hbm, out_hbm, dst_v, acc_v):
    w = _wid()
    pltpu.sync_copy(dst_hbm.at[pl.ds(w * _EPT, _EPT)], dst_v)
    zero16 = jnp.zeros((16,), jnp.float32)

    @pl.loop(0, _N // 16)
    def _zero(i):
        acc_v[pl.ds(i * 16, 16)] = zero16

    one16 = jnp.ones((16,), jnp.float32)

    @pl.loop(0, _EPT // 16)
    def _hist(j):
        idx = dst_v[pl.ds(j * 16, 16)]
        plsc.addupdate_scatter(acc_v, [idx], one16)

    pltpu.sync_copy(acc_v, out_hbm.at[w])


_deg_hist = pl.kernel(
    _deg_body,
    out_type=jax.ShapeDtypeStruct((_NW, _N), jnp.float32),
    mesh=_mesh,
    scratch_types=[
        pltpu.VMEM((_EPT,), jnp.int32),
        pltpu.VMEM((_N,), jnp.float32),
    ],
)


def _segsum_body(g_hbm, src_hbm, dst_hbm, zeros_hbm, out_hbm,
                 sidx, didx, rows, acc_s, sem):
    c = lax.axis_index("c")
    s = lax.axis_index("s")
    w = s * _NC + c
    pltpu.sync_copy(zeros_hbm.at[pl.ds(s * _RPT, _RPT)],
                    acc_s.at[pl.ds(s * _RPT, _RPT)])
    plsc.subcore_barrier()

    @pl.loop(0, _NB)
    def _blk(b):
        off = w * _EPT + b * _EB
        pltpu.sync_copy(src_hbm.at[pl.ds(off, _EB)], sidx)
        pltpu.async_copy(g_hbm.at[sidx], rows, sem).wait()
        pltpu.sync_copy(dst_hbm.at[pl.ds(off, _EB)], didx)
        pltpu.sync_copy(rows, acc_s.at[didx], add=True)

    plsc.subcore_barrier()
    pltpu.sync_copy(acc_s.at[pl.ds(s * _RPT, _RPT)],
                    out_hbm.at[c, pl.ds(s * _RPT, _RPT)])


_segsum = pl.kernel(
    _segsum_body,
    out_type=jax.ShapeDtypeStruct((_NC, _N, _H), jnp.float32),
    mesh=_mesh,
    scratch_types=[
        pltpu.VMEM((_EB,), jnp.int32),
        pltpu.VMEM((_EB,), jnp.int32),
        pltpu.VMEM((_EB, _H), jnp.float32),
        pltpu.VMEM_SHARED((_N, _H), jnp.float32),
        pltpu.SemaphoreType.DMA,
    ],
)


def _edge_gather_body(a_hbm, b_hbm, src_hbm, dst_hbm, outa_hbm, outb_hbm,
                      sidx, didx, rows_a, rows_b, sem_a, sem_b):
    w = _wid()

    @pl.loop(0, _NB)
    def _blk(b):
        off = w * _EPT + b * _EB
        pltpu.sync_copy(src_hbm.at[pl.ds(off, _EB)], sidx)
        cp_a = pltpu.async_copy(a_hbm.at[sidx], rows_a, sem_a)
        pltpu.sync_copy(dst_hbm.at[pl.ds(off, _EB)], didx)
        cp_b = pltpu.async_copy(b_hbm.at[didx], rows_b, sem_b)
        cp_a.wait()
        pltpu.sync_copy(rows_a, outa_hbm.at[pl.ds(off, _EB)])
        cp_b.wait()
        pltpu.sync_copy(rows_b, outb_hbm.at[pl.ds(off, _EB)])


_edge_gather = pl.kernel(
    _edge_gather_body,
    out_type=(jax.ShapeDtypeStruct((_E, _H), jnp.float32),
              jax.ShapeDtypeStruct((_E, _H), jnp.float32)),
    mesh=_mesh,
    scratch_types=[
        pltpu.VMEM((_EB,), jnp.int32),
        pltpu.VMEM((_EB,), jnp.int32),
        pltpu.VMEM((_EB, _H), jnp.float32),
        pltpu.VMEM((_EB, _H), jnp.float32),
        pltpu.SemaphoreType.DMA,
        pltpu.SemaphoreType.DMA,
    ],
)


def _enc_kernel(nf, degt, w1, b1, w2, b2, cw, x_out, g_out, dinv_out):
    deg = jnp.sum(degt[...], axis=1, keepdims=True) + 1.0
    dinv = lax.rsqrt(deg)
    x = jnp.maximum(jnp.dot(nf[...], w1[...],
                            preferred_element_type=jnp.float32) + b1[...], 0.0)
    x = jnp.dot(x, w2[...], preferred_element_type=jnp.float32) + b2[...]
    x_out[...] = x
    dinv_out[...] = dinv
    g_out[...] = jnp.dot(x, cw[...], preferred_element_type=jnp.float32) * dinv


def _mid_kernel(parts, g_prev, dinv, bias, w_next, g_out):
    x = (parts[0] + parts[1] + g_prev[...]) * dinv[...] + bias[...]
    x = jnp.maximum(x, 0.0)
    g_out[...] = jnp.dot(x, w_next[...],
                         preferred_element_type=jnp.float32) * dinv[...]


def _last_kernel(parts, g_prev, dinv, bias, w_top, b_top, w_bot, a_out, b_out):
    x = (parts[0] + parts[1] + g_prev[...]) * dinv[...] + bias[...]
    a_out[...] = jnp.dot(x, w_top[...],
                         preferred_element_type=jnp.float32) + b_top[...]
    b_out[...] = jnp.dot(x, w_bot[...],
                         preferred_element_type=jnp.float32)


def _score_kernel(ga, gb, w2, b2, s_out):
    h = jnp.maximum(ga[...] + gb[...], 0.0)
    s = jnp.dot(h, w2[...], preferred_element_type=jnp.float32) + b2[...]
    s_out[...] = jax.nn.sigmoid(s)


def _full(shape):
    return pl.BlockSpec(shape, lambda i: (0,) * len(shape))


def _rows(shape):
    return pl.BlockSpec(shape, lambda i: (i,) + (0,) * (len(shape) - 1))


_GRID_N = _N // _RB
_GRID_E = _E // _EBT

_enc_call = pl.pallas_call(
    _enc_kernel,
    grid=(_GRID_N,),
    in_specs=[
        _rows((_RB, _H)), _rows((_RB, _NW)),
        _full((_H, _H)), _full((1, _H)), _full((_H, _H)), _full((1, _H)),
        _full((_H, _H)),
    ],
    out_specs=[_rows((_RB, _H)), _rows((_RB, _H)), _rows((_RB, 1))],
    out_shape=[
        jax.ShapeDtypeStruct((_N, _H), jnp.float32),
        jax.ShapeDtypeStruct((_N, _H), jnp.float32),
        jax.ShapeDtypeStruct((_N, 1), jnp.float32),
    ],
)

_mid_call = pl.pallas_call(
    _mid_kernel,
    grid=(_GRID_N,),
    in_specs=[
        pl.BlockSpec((_NC, _RB, _H), lambda i: (0, i, 0)),
        _rows((_RB, _H)), _rows((_RB, 1)), _full((1, _H)), _full((_H, _H)),
    ],
    out_specs=[_rows((_RB, _H))],
    out_shape=[jax.ShapeDtypeStruct((_N, _H), jnp.float32)],
)

_last_call = pl.pallas_call(
    _last_kernel,
    grid=(_GRID_N,),
    in_specs=[
        pl.BlockSpec((_NC, _RB, _H), lambda i: (0, i, 0)),
        _rows((_RB, _H)), _rows((_RB, 1)), _full((1, _H)),
        _full((_H, _H)), _full((1, _H)), _full((_H, _H)),
    ],
    out_specs=[_rows((_RB, _H)), _rows((_RB, _H))],
    out_shape=[
        jax.ShapeDtypeStruct((_N, _H), jnp.float32),
        jax.ShapeDtypeStruct((_N, _H), jnp.float32),
    ],
)

_score_call = pl.pallas_call(
    _score_kernel,
    grid=(_GRID_E,),
    in_specs=[
        _rows((_EBT, _H)), _rows((_EBT, _H)),
        _full((_H, 1)), _full((1, 1)),
    ],
    out_specs=[_rows((_EBT, 1))],
    out_shape=[jax.ShapeDtypeStruct((_E, 1), jnp.float32)],
)


def kernel(node_features, edge_index, enc_w1, enc_b1, enc_w2, enc_b2,
           conv1_w, conv1_b, conv2_w, conv2_b, conv3_w, conv3_b,
           cls_w1, cls_b1, cls_w2, cls_b2):
    src = edge_index[0]
    dst = edge_index[1]
    zeros = jnp.zeros((_N, _H), jnp.float32)

    deg_parts = _deg_hist(dst)
    degt = deg_parts.T

    x, g1, dinv = _enc_call(
        node_features, degt, enc_w1, enc_b1.reshape(1, _H),
        enc_w2, enc_b2.reshape(1, _H), conv1_w)

    p1 = _segsum(g1, src, dst, zeros)
    (g2,) = _mid_call(p1, g1, dinv, conv1_b.reshape(1, _H), conv2_w)

    p2 = _segsum(g2, src, dst, zeros)
    (g3,) = _mid_call(p2, g2, dinv, conv2_b.reshape(1, _H), conv3_w)

    p3 = _segsum(g3, src, dst, zeros)
    a_nodes, b_nodes = _last_call(
        p3, g3, dinv, conv3_b.reshape(1, _H),
        cls_w1[:_H], cls_b1.reshape(1, _H), cls_w1[_H:])

    ga, gb = _edge_gather(a_nodes, b_nodes, src, dst)
    (scores,) = _score_call(ga, gb, cls_w2, cls_b2.reshape(1, 1))
    return scores.reshape(_E)


# speedup vs baseline: 8.1012x; 8.1012x over previous
import jax
import jax.numpy as jnp
from jax import lax
from jax.experimental import pallas as pl
from jax.experimental.pallas import tpu as pltpu
from jax.experimental.pallas import tpu_sc as plsc

_N = 10000
_E = 320000
_H = 128

_NC = 2
_NS = 16
_NW = _NC * _NS
_EPT = _E // _NW
_EB = 80
_NB = _EPT // _EB
_RPT = _N // _NS

_RB = 2000
_EBT = 4000

_mesh = plsc.VectorSubcoreMesh(core_axis_name="c", subcore_axis_name="s")


def _wid():
    return lax.axis_index("s") * _NC + lax.axis_index("c")


def _deg_body(dst_hbm, out_hbm, didx, ones_v, stage_d, acc_s):
    c = lax.axis_index("c")
    s = lax.axis_index("s")
    w = s * _NC + c
    zero16 = jnp.zeros((16,), jnp.float32)
    one16 = jnp.ones((16,), jnp.float32)
    @pl.when(s < 5)
    def _z():
        @pl.loop(0, 2000 // 16)
        def _f(i):
            stage_d[pl.ds(i * 16, 16)] = zero16
        pltpu.sync_copy(stage_d, acc_s.at[pl.ds(s * 2000, 2000)])

    for i in range(_EB // 16):
        ones_v[pl.ds(i * 16, 16)] = one16
    plsc.subcore_barrier()

    @pl.loop(0, _NB)
    def _blk(b):
        off = w * _EPT + b * _EB
        pltpu.sync_copy(dst_hbm.at[pl.ds(off, _EB)], didx)
        pltpu.sync_copy(ones_v, acc_s.at[didx], add=True)

    plsc.subcore_barrier()

    @pl.when(s < 5)
    def _w():
        pltpu.sync_copy(acc_s.at[pl.ds(s * 2000, 2000)], stage_d)
        pltpu.sync_copy(stage_d, out_hbm.at[pl.ds(c * _N + s * 2000, 2000)])


_deg_hist = pl.kernel(
    _deg_body,
    out_type=jax.ShapeDtypeStruct((_NC * _N,), jnp.float32),
    mesh=_mesh,
    scratch_types=[
        pltpu.VMEM((_EB,), jnp.int32),
        pltpu.VMEM((_EB,), jnp.float32),
        pltpu.VMEM((2000,), jnp.float32),
        pltpu.VMEM_SHARED((_N,), jnp.float32),
    ],
)


_ZR = 200


def _segsum_body(g_hbm, src_hbm, dst_hbm, zeros_hbm, out_hbm,
                 sidx, didx, rows, stage, acc_s, sem):
    c = lax.axis_index("c")
    s = lax.axis_index("s")
    w = s * _NC + c
    @pl.when(s < 10)
    def _z():
        pltpu.sync_copy(zeros_hbm.at[pl.ds(0, _ZR)], stage)

        @pl.loop(0, 1000 // _ZR)
        def _zz(k):
            pltpu.sync_copy(stage, acc_s.at[pl.ds(s * 1000 + k * _ZR, _ZR)])

    plsc.subcore_barrier()

    @pl.loop(0, _NB)
    def _blk(b):
        off = w * _EPT + b * _EB
        pltpu.sync_copy(src_hbm.at[pl.ds(off, _EB)], sidx)
        pltpu.async_copy(g_hbm.at[sidx], rows, sem).wait()
        pltpu.sync_copy(dst_hbm.at[pl.ds(off, _EB)], didx)
        pltpu.sync_copy(rows, acc_s.at[didx], add=True)

    plsc.subcore_barrier()

    @pl.when(s < 10)
    def _w():
        @pl.loop(0, 1000 // _ZR)
        def _ww(k):
            off = s * 1000 + k * _ZR
            pltpu.sync_copy(acc_s.at[pl.ds(off, _ZR)], stage)
            pltpu.sync_copy(stage, out_hbm.at[c, pl.ds(off, _ZR)])


_segsum = pl.kernel(
    _segsum_body,
    out_type=jax.ShapeDtypeStruct((_NC, _N, _H), jnp.float32),
    mesh=_mesh,
    scratch_types=[
        pltpu.VMEM((_EB,), jnp.int32),
        pltpu.VMEM((_EB,), jnp.int32),
        pltpu.VMEM((_EB, _H), jnp.float32),
        pltpu.VMEM((_ZR, _H), jnp.float32),
        pltpu.VMEM_SHARED((_N, _H), jnp.float32),
        pltpu.SemaphoreType.DMA,
    ],
)


def _edge_gather_body(a_hbm, b_hbm, src_hbm, dst_hbm, outa_hbm, outb_hbm,
                      sidx, didx, rows_a, rows_b, sem_a, sem_b):
    w = _wid()

    @pl.loop(0, _NB)
    def _blk(b):
        off = w * _EPT + b * _EB
        pltpu.sync_copy(src_hbm.at[pl.ds(off, _EB)], sidx)
        cp_a = pltpu.async_copy(a_hbm.at[sidx], rows_a, sem_a)
        pltpu.sync_copy(dst_hbm.at[pl.ds(off, _EB)], didx)
        cp_b = pltpu.async_copy(b_hbm.at[didx], rows_b, sem_b)
        cp_a.wait()
        pltpu.sync_copy(rows_a, outa_hbm.at[pl.ds(off, _EB)])
        cp_b.wait()
        pltpu.sync_copy(rows_b, outb_hbm.at[pl.ds(off, _EB)])


_edge_gather = pl.kernel(
    _edge_gather_body,
    out_type=(jax.ShapeDtypeStruct((_E, _H), jnp.float32),
              jax.ShapeDtypeStruct((_E, _H), jnp.float32)),
    mesh=_mesh,
    scratch_types=[
        pltpu.VMEM((_EB,), jnp.int32),
        pltpu.VMEM((_EB,), jnp.int32),
        pltpu.VMEM((_EB, _H), jnp.float32),
        pltpu.VMEM((_EB, _H), jnp.float32),
        pltpu.SemaphoreType.DMA,
        pltpu.SemaphoreType.DMA,
    ],
)


def _enc_kernel(nf, degt, w1, b1, w2, b2, cw, x_out, g_out, dinv_out):
    deg = jnp.sum(degt[...], axis=1, keepdims=True) + 1.0
    dinv = lax.rsqrt(deg)
    x = jnp.maximum(jnp.dot(nf[...], w1[...],
                            preferred_element_type=jnp.float32) + b1[...], 0.0)
    x = jnp.dot(x, w2[...], preferred_element_type=jnp.float32) + b2[...]
    x_out[...] = x
    dinv_out[...] = dinv
    g_out[...] = jnp.dot(x, cw[...], preferred_element_type=jnp.float32) * dinv


def _mid_kernel(parts, g_prev, dinv, bias, w_next, g_out):
    x = (parts[0] + parts[1] + g_prev[...]) * dinv[...] + bias[...]
    x = jnp.maximum(x, 0.0)
    g_out[...] = jnp.dot(x, w_next[...],
                         preferred_element_type=jnp.float32) * dinv[...]


def _last_kernel(parts, g_prev, dinv, bias, w_top, b_top, w_bot, a_out, b_out):
    x = (parts[0] + parts[1] + g_prev[...]) * dinv[...] + bias[...]
    a_out[...] = jnp.dot(x, w_top[...],
                         preferred_element_type=jnp.float32) + b_top[...]
    b_out[...] = jnp.dot(x, w_bot[...],
                         preferred_element_type=jnp.float32)


def _score_kernel(ga, gb, w2, b2, s_out):
    h = jnp.maximum(ga[...] + gb[...], 0.0)
    s = jnp.dot(h, w2[...], preferred_element_type=jnp.float32) + b2[...]
    s_out[...] = jax.nn.sigmoid(s)


def _full(shape):
    return pl.BlockSpec(shape, lambda i: (0,) * len(shape))


def _rows(shape):
    return pl.BlockSpec(shape, lambda i: (i,) + (0,) * (len(shape) - 1))


_GRID_N = _N // _RB
_GRID_E = _E // _EBT

_enc_call = pl.pallas_call(
    _enc_kernel,
    grid=(_GRID_N,),
    in_specs=[
        _rows((_RB, _H)), _rows((_RB, _NC)),
        _full((_H, _H)), _full((1, _H)), _full((_H, _H)), _full((1, _H)),
        _full((_H, _H)),
    ],
    out_specs=[_rows((_RB, _H)), _rows((_RB, _H)), _rows((_RB, 1))],
    out_shape=[
        jax.ShapeDtypeStruct((_N, _H), jnp.float32),
        jax.ShapeDtypeStruct((_N, _H), jnp.float32),
        jax.ShapeDtypeStruct((_N, 1), jnp.float32),
    ],
)

_mid_call = pl.pallas_call(
    _mid_kernel,
    grid=(_GRID_N,),
    in_specs=[
        pl.BlockSpec((_NC, _RB, _H), lambda i: (0, i, 0)),
        _rows((_RB, _H)), _rows((_RB, 1)), _full((1, _H)), _full((_H, _H)),
    ],
    out_specs=[_rows((_RB, _H))],
    out_shape=[jax.ShapeDtypeStruct((_N, _H), jnp.float32)],
)

_last_call = pl.pallas_call(
    _last_kernel,
    grid=(_GRID_N,),
    in_specs=[
        pl.BlockSpec((_NC, _RB, _H), lambda i: (0, i, 0)),
        _rows((_RB, _H)), _rows((_RB, 1)), _full((1, _H)),
        _full((_H, _H)), _full((1, _H)), _full((_H, _H)),
    ],
    out_specs=[_rows((_RB, _H)), _rows((_RB, _H))],
    out_shape=[
        jax.ShapeDtypeStruct((_N, _H), jnp.float32),
        jax.ShapeDtypeStruct((_N, _H), jnp.float32),
    ],
)

_score_call = pl.pallas_call(
    _score_kernel,
    grid=(_GRID_E,),
    in_specs=[
        _rows((_EBT, _H)), _rows((_EBT, _H)),
        _full((_H, 1)), _full((1, 1)),
    ],
    out_specs=[_rows((_EBT, 1))],
    out_shape=[jax.ShapeDtypeStruct((_E, 1), jnp.float32)],
)


def kernel(node_features, edge_index, enc_w1, enc_b1, enc_w2, enc_b2,
           conv1_w, conv1_b, conv2_w, conv2_b, conv3_w, conv3_b,
           cls_w1, cls_b1, cls_w2, cls_b2):
    src = edge_index[0]
    dst = edge_index[1]
    zeros = jnp.zeros((_N, _H), jnp.float32)

    deg_parts = _deg_hist(dst)
    degt = deg_parts.reshape(_NC, _N).T

    x, g1, dinv = _enc_call(
        node_features, degt, enc_w1, enc_b1.reshape(1, _H),
        enc_w2, enc_b2.reshape(1, _H), conv1_w)

    p1 = _segsum(g1, src, dst, zeros)
    (g2,) = _mid_call(p1, g1, dinv, conv1_b.reshape(1, _H), conv2_w)

    p2 = _segsum(g2, src, dst, zeros)
    (g3,) = _mid_call(p2, g2, dinv, conv2_b.reshape(1, _H), conv3_w)

    p3 = _segsum(g3, src, dst, zeros)
    a_nodes, b_nodes = _last_call(
        p3, g3, dinv, conv3_b.reshape(1, _H),
        cls_w1[:_H], cls_b1.reshape(1, _H), cls_w1[_H:])

    ga, gb = _edge_gather(a_nodes, b_nodes, src, dst)
    (scores,) = _score_call(ga, gb, cls_w2, cls_b2.reshape(1, 1))
    return scores.reshape(_E)
